# baseline (speedup 1.0000x reference)
import jax
import jax.numpy as jnp
from jax.experimental import pallas as pl
from jax.experimental.pallas import tpu as pltpu

N = 10000
F_IN = 128
H = 64
C = 32
BI = 200
NI = N // BI
NBUF = 5


def _stripe(i):
    return jnp.where(i < NI, i, 2 * NI - 1 - i)


def _gcn_body(adj_hbm, x_ref, W1_ref, b1_ref, W2_ref, b2_ref, out_ref,
              s1_ref, s2_ref):
    s1_ref[...] = jnp.dot(x_ref[...], W1_ref[...],
                          preferred_element_type=jnp.float32)

    def inner(idx, stripe_ref):
        i = idx[0]

        @pl.when(i < NI)
        def _():
            g = jnp.dot(stripe_ref[...], s1_ref[...],
                        preferred_element_type=jnp.float32)
            h = jnp.maximum(g + b1_ref[...], 0.0)
            s2_ref[pl.ds(i * BI, BI), :] = jnp.dot(
                h, W2_ref[...], preferred_element_type=jnp.float32)

        @pl.when(i >= NI)
        def _():
            acc = jnp.dot(stripe_ref[...], s2_ref[...],
                          preferred_element_type=jnp.float32)
            out_ref[pl.ds((2 * NI - 1 - i) * BI, BI), :] = acc + b2_ref[...]

    pipe = pltpu.emit_pipeline(
        inner,
        grid=(2 * NI,),
        in_specs=[
            pl.BlockSpec((BI, N), lambda i: (_stripe(i), 0),
                         pipeline_mode=pl.Buffered(buffer_count=NBUF,
                                                   use_lookahead=True)),
        ],
        _explicit_indices=True,
    )
    pipe(adj_hbm)


def kernel(x, adj, W1, b1, W2, b2):
    b1r = b1.reshape(1, H)
    b2r = b2.reshape(1, C)
    out = pl.pallas_call(
        _gcn_body,
        in_specs=[
            pl.BlockSpec(memory_space=pltpu.MemorySpace.HBM),
            pl.BlockSpec(memory_space=pltpu.VMEM),
            pl.BlockSpec(memory_space=pltpu.VMEM),
            pl.BlockSpec(memory_space=pltpu.VMEM),
            pl.BlockSpec(memory_space=pltpu.VMEM),
            pl.BlockSpec(memory_space=pltpu.VMEM),
        ],
        out_specs=pl.BlockSpec(memory_space=pltpu.VMEM),
        out_shape=jax.ShapeDtypeStruct((N, C), jnp.float32),
        scratch_shapes=[
            pltpu.VMEM((N, H), jnp.float32),
            pltpu.VMEM((N, C), jnp.float32),
        ],
    )(adj, x, W1, b1r, W2, b2r)
    return out


# speedup vs baseline: 1.0014x; 1.0014x over previous
import jax
import jax.numpy as jnp
from jax.experimental import pallas as pl
from jax.experimental.pallas import tpu as pltpu

N = 10000
F_IN = 128
H = 64
C = 32
BI = 200
NI = N // BI
NBUF = 5


def _stripe(i):
    return jnp.where(i < NI, i, 2 * NI - 1 - i)


def _gcn_body(adj_hbm, x_hbm, W1_ref, b1_ref, W2_ref, b2_ref, out_ref,
              x_vmem, s1_ref, s2_ref, xsem):
    xcopy = pltpu.make_async_copy(x_hbm, x_vmem, xsem)
    xcopy.start()

    def inner(idx, stripe_ref):
        i = idx[0]

        @pl.when(i == 0)
        def _():
            xcopy.wait()
            s1_ref[...] = jnp.dot(x_vmem[...], W1_ref[...],
                                  preferred_element_type=jnp.float32)

        @pl.when(i < NI)
        def _():
            g = jnp.dot(stripe_ref[...], s1_ref[...],
                        preferred_element_type=jnp.float32)
            h = jnp.maximum(g + b1_ref[...], 0.0)
            s2_ref[pl.ds(i * BI, BI), :] = jnp.dot(
                h, W2_ref[...], preferred_element_type=jnp.float32)

        @pl.when(i >= NI)
        def _():
            acc = jnp.dot(stripe_ref[...], s2_ref[...],
                          preferred_element_type=jnp.float32)
            out_ref[pl.ds((2 * NI - 1 - i) * BI, BI), :] = acc + b2_ref[...]

    pipe = pltpu.emit_pipeline(
        inner,
        grid=(2 * NI,),
        in_specs=[
            pl.BlockSpec((BI, N), lambda i: (_stripe(i), 0),
                         pipeline_mode=pl.Buffered(buffer_count=NBUF,
                                                   use_lookahead=True)),
        ],
        _explicit_indices=True,
    )
    pipe(adj_hbm)


def kernel(x, adj, W1, b1, W2, b2):
    b1r = b1.reshape(1, H)
    b2r = b2.reshape(1, C)
    out = pl.pallas_call(
        _gcn_body,
        in_specs=[
            pl.BlockSpec(memory_space=pltpu.MemorySpace.HBM),
            pl.BlockSpec(memory_space=pltpu.MemorySpace.HBM),
            pl.BlockSpec(memory_space=pltpu.VMEM),
            pl.BlockSpec(memory_space=pltpu.VMEM),
            pl.BlockSpec(memory_space=pltpu.VMEM),
            pl.BlockSpec(memory_space=pltpu.VMEM),
        ],
        out_specs=pl.BlockSpec(memory_space=pltpu.VMEM),
        out_shape=jax.ShapeDtypeStruct((N, C), jnp.float32),
        scratch_shapes=[
            pltpu.VMEM((N, F_IN), jnp.float32),
            pltpu.VMEM((N, H), jnp.float32),
            pltpu.VMEM((N, C), jnp.float32),
            pltpu.SemaphoreType.DMA,
        ],
    )(adj, x, W1, b1r, W2, b2r)
    return out
